# baseline (speedup 1.0000x reference)
import functools

import jax
import jax.numpy as jnp
from jax import lax
from jax.experimental import pallas as pl
from jax.experimental.pallas import tpu as pltpu
from jax.experimental.pallas import tpu_sc as plsc

NC = 2
NS = 16
NW = NC * NS
LANES = 16

BLOCK = 200
IDX_MINOR = 100


def kernel(token_ids, table, pos_emb):
    B, S = token_ids.shape
    V, H = table.shape
    total = B * S
    assert total % NW == 0
    rows_per_w = total // NW
    assert rows_per_w % BLOCK == 0
    blocks_per_w = rows_per_w // BLOCK
    idx_rows_per_block = BLOCK // IDX_MINOR
    idx_rows_per_w = rows_per_w // IDX_MINOR

    idx2d = token_ids.reshape(total // IDX_MINOR, IDX_MINOR)

    mesh = plsc.VectorSubcoreMesh(core_axis_name="c", subcore_axis_name="s")

    @functools.partial(
        pl.kernel,
        out_type=jax.ShapeDtypeStruct((total, H), jnp.float32),
        mesh=mesh,
        compiler_params=pltpu.CompilerParams(use_tc_tiling_on_sc=False),
        scratch_types=[
            pltpu.VMEM((idx_rows_per_w, IDX_MINOR), jnp.int32),
            pltpu.VMEM((BLOCK, H), jnp.float32),
            pltpu.VMEM((S, H), jnp.float32),
            pltpu.SemaphoreType.DMA,
        ],
    )
    def emb(idx_hbm, pos_hbm, table_hbm, out_hbm, idx_v, rows_v, pos_v, sem):
        wid = lax.axis_index("s") * NC + lax.axis_index("c")
        pltpu.sync_copy(pos_hbm, pos_v)
        base_row = wid * rows_per_w
        pltpu.sync_copy(idx_hbm.at[pl.ds(wid * idx_rows_per_w, idx_rows_per_w)],
                        idx_v)

        def block_body(blk, carry):
            row0 = base_row + blk * BLOCK
            copies = [
                pltpu.async_copy(
                    table_hbm.at[idx_v.at[blk * idx_rows_per_block + j]],
                    rows_v.at[pl.ds(j * IDX_MINOR, IDX_MINOR)],
                    sem,
                )
                for j in range(idx_rows_per_block)
            ]
            for cp in copies:
                cp.wait()

            def add_body(r, c2):
                for q in range(H // LANES):
                    sl = pl.ds(q * LANES, LANES)
                    plsc.addupdate(rows_v.at[r, sl], pos_v[r, sl])
                return c2

            lax.fori_loop(0, BLOCK, add_body, 0)
            pltpu.sync_copy(rows_v, out_hbm.at[pl.ds(row0, BLOCK)])
            return carry

        lax.fori_loop(0, blocks_per_w, block_body, 0)

    out = emb(idx2d, pos_emb, table)
    return out.reshape(B, S, H)


# speedup vs baseline: 1.1562x; 1.1562x over previous
import functools

import jax
import jax.numpy as jnp
from jax import lax
from jax.experimental import pallas as pl
from jax.experimental.pallas import tpu as pltpu
from jax.experimental.pallas import tpu_sc as plsc

NC = 2
NS = 16
NW = NC * NS
LANES = 16

SEQ = 200
BLOCK = 2 * SEQ
IDX_MINOR = 100
IPB = BLOCK // IDX_MINOR


def kernel(token_ids, table, pos_emb):
    B, S = token_ids.shape
    V, H = table.shape
    total = B * S
    assert S == SEQ and H == 4 * LANES
    rows_per_w = total // NW
    nblocks = rows_per_w // BLOCK
    idx_rows_per_w = rows_per_w // IDX_MINOR
    assert nblocks % 2 == 0 and nblocks >= 4

    idx2d = token_ids.reshape(total // IDX_MINOR, IDX_MINOR)

    mesh = plsc.VectorSubcoreMesh(core_axis_name="c", subcore_axis_name="s")

    @functools.partial(
        pl.kernel,
        out_type=jax.ShapeDtypeStruct((total, H), jnp.float32),
        mesh=mesh,
        compiler_params=pltpu.CompilerParams(use_tc_tiling_on_sc=False),
        scratch_types=[
            pltpu.VMEM((idx_rows_per_w, IDX_MINOR), jnp.int32),
            pltpu.VMEM((BLOCK, H), jnp.float32),
            pltpu.VMEM((BLOCK, H), jnp.float32),
            pltpu.VMEM((S, H), jnp.float32),
            pltpu.SemaphoreType.DMA,
            pltpu.SemaphoreType.DMA,
            pltpu.SemaphoreType.DMA,
            pltpu.SemaphoreType.DMA,
        ],
    )
    def emb(idx_hbm, pos_hbm, table_hbm, out_hbm,
            idx_v, buf0, buf1, pos_v, gsem0, gsem1, wsem0, wsem1):
        bufs = (buf0, buf1)
        gsems = (gsem0, gsem1)
        wsems = (wsem0, wsem1)
        wid = lax.axis_index("s") * NC + lax.axis_index("c")
        base_row = wid * rows_per_w
        pltpu.sync_copy(pos_hbm, pos_v)
        pltpu.sync_copy(idx_hbm.at[pl.ds(wid * idx_rows_per_w, idx_rows_per_w)],
                        idx_v)

        def issue_gathers(g, b):
            for j in range(IPB):
                pltpu.async_copy(
                    table_hbm.at[idx_v.at[g * IPB + j]],
                    bufs[b].at[pl.ds(j * IDX_MINOR, IDX_MINOR)],
                    gsems[b],
                )

        def wait_gathers(b):
            pltpu.make_async_copy(
                table_hbm.at[pl.ds(0, BLOCK)], bufs[b], gsems[b]).wait()

        def issue_write(g, b):
            pltpu.async_copy(
                bufs[b], out_hbm.at[pl.ds(base_row + g * BLOCK, BLOCK)],
                wsems[b])

        def wait_write(b):
            pltpu.make_async_copy(
                bufs[b], out_hbm.at[pl.ds(0, BLOCK)], wsems[b]).wait()

        def add_pos(b):
            buf = bufs[b]

            def add_body(r, c):
                for q in range(H // LANES):
                    sl = pl.ds(q * LANES, LANES)
                    pv = pos_v[r, sl]
                    for s in range(BLOCK // SEQ):
                        plsc.addupdate(buf.at[s * SEQ + r, sl], pv)
                return c

            lax.fori_loop(0, SEQ, add_body, 0)

        issue_gathers(0, 0)
        issue_gathers(1, 1)
        wait_gathers(0)
        add_pos(0)
        issue_write(0, 0)

        def step(g, b):
            wait_gathers(b)
            wait_write(b ^ 1)
            issue_gathers(g + 1, b ^ 1)
            add_pos(b)
            issue_write(g, b)

        def pair(i, c):
            step(2 * i + 1, 1)
            step(2 * i + 2, 0)
            return c

        lax.fori_loop(0, (nblocks - 2) // 2, pair, 0)

        wait_gathers(1)
        wait_write(0)
        add_pos(1)
        issue_write(nblocks - 1, 1)
        wait_write(1)

    out = emb(idx2d, pos_emb, table)
    return out.reshape(B, S, H)


# speedup vs baseline: 1.1575x; 1.0011x over previous
import functools

import jax
import jax.numpy as jnp
from jax import lax
from jax.experimental import pallas as pl
from jax.experimental.pallas import tpu as pltpu
from jax.experimental.pallas import tpu_sc as plsc

NC = 2
NS = 16
NW = NC * NS
LANES = 16

SPB = 2
CHUNKS = ((0, 128), (128, 72))


def kernel(token_ids, table, pos_emb):
    B, S = token_ids.shape
    V, H = table.shape
    assert H == 4 * LANES and sum(c for _, c in CHUNKS) == S
    seqs_per_w = B // NW
    nblocks = seqs_per_w // SPB
    assert nblocks % 2 == 0 and nblocks >= 4

    mesh = plsc.VectorSubcoreMesh(core_axis_name="c", subcore_axis_name="s")

    @functools.partial(
        pl.kernel,
        out_type=jax.ShapeDtypeStruct((B, S, H), jnp.float32),
        mesh=mesh,
        compiler_params=pltpu.CompilerParams(use_tc_tiling_on_sc=False),
        scratch_types=[
            pltpu.VMEM((seqs_per_w, S), jnp.int32),
            pltpu.VMEM((SPB, S, H), jnp.float32),
            pltpu.VMEM((SPB, S, H), jnp.float32),
            pltpu.VMEM((S, H), jnp.float32),
            pltpu.SemaphoreType.DMA,
            pltpu.SemaphoreType.DMA,
            pltpu.SemaphoreType.DMA,
            pltpu.SemaphoreType.DMA,
        ],
    )
    def emb(idx_hbm, pos_hbm, table_hbm, out_hbm,
            idx_v, buf0, buf1, pos_v, gsem0, gsem1, wsem0, wsem1):
        bufs = (buf0, buf1)
        gsems = (gsem0, gsem1)
        wsems = (wsem0, wsem1)
        wid = lax.axis_index("s") * NC + lax.axis_index("c")
        base_seq = wid * seqs_per_w
        pltpu.sync_copy(pos_hbm, pos_v)
        pltpu.sync_copy(idx_hbm.at[pl.ds(base_seq, seqs_per_w)], idx_v)

        def issue_gathers(g, b):
            for s in range(SPB):
                for off, cnt in CHUNKS:
                    pltpu.async_copy(
                        table_hbm.at[idx_v.at[g * SPB + s, pl.ds(off, cnt)]],
                        bufs[b].at[s].at[pl.ds(off, cnt)],
                        gsems[b],
                    )

        def wait_gathers(b):
            pltpu.make_async_copy(
                out_hbm.at[pl.ds(0, SPB)], bufs[b], gsems[b]).wait()

        def issue_write(g, b):
            pltpu.async_copy(
                bufs[b], out_hbm.at[pl.ds(base_seq + g * SPB, SPB)], wsems[b])

        def wait_write(b):
            pltpu.make_async_copy(
                bufs[b], out_hbm.at[pl.ds(0, SPB)], wsems[b]).wait()

        def add_pos(b):
            buf = bufs[b]

            def add_body(r, c):
                for q in range(H // LANES):
                    sl = pl.ds(q * LANES, LANES)
                    pv = pos_v[r, sl]
                    for s in range(SPB):
                        plsc.addupdate(buf.at[s, r, sl], pv)
                return c

            lax.fori_loop(0, S, add_body, 0)

        issue_gathers(0, 0)
        issue_gathers(1, 1)
        wait_gathers(0)
        add_pos(0)
        issue_write(0, 0)

        def step(g, b):
            wait_gathers(b)
            wait_write(b ^ 1)
            issue_gathers(g + 1, b ^ 1)
            add_pos(b)
            issue_write(g, b)

        def pair(i, c):
            step(2 * i + 1, 1)
            step(2 * i + 2, 0)
            return c

        lax.fori_loop(0, (nblocks - 2) // 2, pair, 0)

        wait_gathers(1)
        wait_write(0)
        add_pos(1)
        issue_write(nblocks - 1, 1)
        wait_write(1)

    return emb(token_ids, pos_emb, table)
